# baseline (speedup 1.0000x reference)
import functools
import jax
import jax.numpy as jnp
from jax import lax
from jax.experimental import pallas as pl
from jax.experimental.pallas import tpu as pltpu
from jax.experimental.pallas import tpu_sc as plsc

BATCH = 16384
N_UNIT = 1024
N_BIN = 50
N_PAD = 64
NC = 2
NS = 16
NW = NC * NS
N_COLG = 8
N_ROWQ = NW // N_COLG
U_PER_W = N_UNIT // N_COLG
B_PER_W = BATCH // N_ROWQ
CHUNK = 256
L = 16
NCELL = 256
SC_ROWS = 9216
SC_B_PER_W = SC_ROWS // N_ROWQ


def _tables_block(xs_ref, ys_ref, lut_ref, inv_ref, dy_ref):
    cells = jax.lax.broadcasted_iota(jnp.int32, (1, NCELL), 1).astype(jnp.float32)
    e_lo = cells * jnp.float32(1.0 / NCELL)
    e_hi = (cells + 1.0) * jnp.float32(1.0 / NCELL)
    cnt_lo = jnp.zeros((N_UNIT, NCELL), jnp.int32)
    cnt_hi = jnp.zeros((N_UNIT, NCELL), jnp.int32)
    one = jnp.int32(1)
    zero = jnp.int32(0)
    for j in range(N_BIN):
        xj = xs_ref[:, j][:, None]
        cnt_lo = cnt_lo + jnp.where(xj <= e_lo, one, zero)
        cnt_hi = cnt_hi + jnp.where(xj <= e_hi, one, zero)
    ulocal = jax.lax.broadcasted_iota(jnp.int32, (N_UNIT, 1), 0) % U_PER_W
    start = ulocal * N_PAD + jnp.maximum(cnt_lo, 1)
    lut_ref[...] = start | ((cnt_hi - cnt_lo) << 16)

    xs = xs_ref[...]
    ys = ys_ref[...]
    x_hi = jnp.concatenate([xs[:, 1:], xs[:, N_BIN - 1:]], axis=1)
    y_hi = jnp.concatenate([ys[:, 1:], ys[:, N_BIN - 1:]], axis=1)
    inv_ref[...] = 1.0 / jnp.maximum(x_hi - xs, jnp.float32(1e-12))
    dy_ref[...] = y_hi - ys


def _build_tables(xs, ys):
    return pl.pallas_call(
        _tables_block,
        out_shape=(
            jax.ShapeDtypeStruct((N_UNIT, NCELL), jnp.int32),
            jax.ShapeDtypeStruct((N_UNIT, N_BIN), jnp.float32),
            jax.ShapeDtypeStruct((N_UNIT, N_BIN), jnp.float32),
        ),
    )(xs, ys)


def _next_down(v):
    bits = jax.lax.bitcast_convert_type(v, jnp.int32)
    dec = jax.lax.bitcast_convert_type(bits - 1, jnp.float32)
    neg_tiny = jnp.float32(-1e-30)
    return jnp.where(v > 0, dec, jnp.minimum(v, neg_tiny) * jnp.float32(1.0000001))


def _isotonic_block(x_ref, xs_ref, ys_ref, o_ref, *, n_bin):
    x = x_ref[...]
    xs_rows = [xs_ref[j, :] for j in range(n_bin)]
    ys_rows = [ys_ref[j, :] for j in range(n_bin)]
    u = [None] * n_bin
    u[n_bin - 1] = xs_rows[n_bin - 1]
    for j in range(n_bin - 2, -1, -1):
        u[j] = jnp.minimum(xs_rows[j], _next_down(u[j + 1]))
    acc = jnp.broadcast_to(ys_rows[0][None, :], x.shape)
    for j in range(n_bin - 1):
        w = u[j + 1] - u[j]
        s = (ys_rows[j + 1] - ys_rows[j]) / w
        t = jnp.minimum(jnp.maximum(x - u[j][None, :], 0.0), w[None, :])
        acc = acc + t * s[None, :]
    lo_mask = x <= xs_rows[0][None, :]
    hi_mask = x >= xs_rows[n_bin - 1][None, :]
    out = jnp.where(lo_mask, ys_rows[0][None, :],
                    jnp.where(hi_mask, ys_rows[n_bin - 1][None, :], acc))
    o_ref[...] = out


def _tc_calibrate(inputs_full, xs_t, ys_t):
    bb = 1024
    off = SC_ROWS // bb
    return pl.pallas_call(
        functools.partial(_isotonic_block, n_bin=N_BIN),
        grid=((BATCH - SC_ROWS) // bb,),
        in_specs=[
            pl.BlockSpec((bb, N_UNIT), lambda i: (i + off, 0)),
            pl.BlockSpec((N_BIN, N_UNIT), lambda i: (0, 0)),
            pl.BlockSpec((N_BIN, N_UNIT), lambda i: (0, 0)),
        ],
        out_specs=pl.BlockSpec((bb, N_UNIT), lambda i: (i, 0)),
        out_shape=jax.ShapeDtypeStruct((BATCH - SC_ROWS, N_UNIT), jnp.float32),
    )(inputs_full, xs_t, ys_t)


def _sc_body(in_hbm, xsp_hbm, ys_hbm, lut_hbm, inv_hbm, dy_hbm, out_hbm,
             xs_v, ys_v, lut_v, inv_v, dy_v, inb, outb):
    wid = lax.axis_index("s") * NC + lax.axis_index("c")
    u0 = (wid // N_ROWQ) * U_PER_W
    r0_base = (wid % N_ROWQ) * SC_B_PER_W

    pltpu.sync_copy(xsp_hbm.at[pl.ds(u0 * N_PAD, U_PER_W * N_PAD)], xs_v)
    pltpu.sync_copy(ys_hbm.at[pl.ds(u0 * N_BIN, U_PER_W * N_BIN)], ys_v)
    pltpu.sync_copy(lut_hbm.at[pl.ds(u0 * NCELL, U_PER_W * NCELL)], lut_v)
    pltpu.sync_copy(inv_hbm.at[pl.ds(u0 * N_BIN, U_PER_W * N_BIN)], inv_v)
    pltpu.sync_copy(dy_hbm.at[pl.ds(u0 * N_BIN, U_PER_W * N_BIN)], dy_v)

    lane = lax.iota(jnp.int32, L)
    lane_x = lane * N_PAD
    lane_l = lane * NCELL
    lane_y = lane * N_BIN
    n_h = U_PER_W // L

    lane_d = lane * (N_BIN - N_PAD)

    def _finish(h, x, rf, xbase, clip_low):
        if clip_low:
            rf = jnp.maximum(rf, xbase + 1)
        xlo_i = jnp.minimum(rf - 1, xbase + (N_BIN - 2))
        ylo_i = xlo_i + (lane_d + h * L * (N_BIN - N_PAD))
        x_lo = plsc.load_gather(xs_v, [xlo_i])
        y_lo = plsc.load_gather(ys_v, [ylo_i])
        inv = plsc.load_gather(inv_v, [ylo_i])
        dy = plsc.load_gather(dy_v, [ylo_i])
        t = jnp.clip((x - x_lo) * inv, jnp.float32(0.0), jnp.float32(1.0))
        return y_lo + t * dy

    zero16 = jnp.zeros((L,), jnp.int32)

    def fast_row(row, flagacc):
        for h in range(n_h):
            x = inb[row, pl.ds(h * L, L)]
            xbase = lane_x + h * (L * N_PAD)
            cell = (x * jnp.float32(NCELL)).astype(jnp.int32)
            lw = plsc.load_gather(lut_v, [lane_l + h * (L * NCELL) + cell])
            rf = lw & 0xFFFF
            gap = lw >> 16
            for step in (4, 2, 1):
                probe = plsc.load_gather(xs_v, [rf + (step - 1)])
                rf = jnp.where(probe <= x, rf + step, rf)
            flagacc = jnp.maximum(flagacc, gap)
            outb[row, pl.ds(h * L, L)] = _finish(h, x, rf, xbase, False)
        return flagacc

    def slow_row(row):
        for h in range(n_h):
            x = inb[row, pl.ds(h * L, L)]
            xbase = lane_x + h * (L * N_PAD)
            rf = xbase
            for step in (32, 16, 8, 4, 2, 1):
                probe = plsc.load_gather(xs_v, [rf + (step - 1)])
                rf = jnp.where(probe <= x, rf + step, rf)
            outb[row, pl.ds(h * L, L)] = _finish(h, x, rf, xbase, True)

    def chunk_body(i, _):
        row0 = r0_base + i * CHUNK
        pltpu.sync_copy(in_hbm.at[pl.ds(row0, CHUNK), pl.ds(u0, U_PER_W)], inb)
        flags = plsc.parallel_loop(0, CHUNK, step=1, unroll=2,
                                   carry=zero16)(fast_row)
        flag_s = jnp.max(flags)

        @pl.when(flag_s > 7)
        def _():
            plsc.parallel_loop(0, CHUNK, step=1, unroll=2)(slow_row)

        pltpu.sync_copy(outb, out_hbm.at[pl.ds(row0, CHUNK), pl.ds(u0, U_PER_W)])
        return _

    lax.fori_loop(0, SC_B_PER_W // CHUNK, chunk_body, None)


@jax.jit
def kernel(inputs, xs, ys):
    xs_pad = jnp.pad(xs, ((0, 0), (0, N_PAD - N_BIN)),
                     constant_values=jnp.finfo(jnp.float32).max)
    lut, inv, dy = _build_tables(xs, ys)
    mesh = plsc.VectorSubcoreMesh(core_axis_name="c", subcore_axis_name="s")
    sc = pl.kernel(
        _sc_body,
        out_type=jax.ShapeDtypeStruct((SC_ROWS, N_UNIT), jnp.float32),
        mesh=mesh,
        scratch_types=[
            pltpu.VMEM((U_PER_W * N_PAD,), jnp.float32),
            pltpu.VMEM((U_PER_W * N_BIN,), jnp.float32),
            pltpu.VMEM((U_PER_W * NCELL,), jnp.int32),
            pltpu.VMEM((U_PER_W * N_BIN,), jnp.float32),
            pltpu.VMEM((U_PER_W * N_BIN,), jnp.float32),
            pltpu.VMEM((CHUNK, U_PER_W), jnp.float32),
            pltpu.VMEM((CHUNK, U_PER_W), jnp.float32),
        ],
        compiler_params=pltpu.CompilerParams(needs_layout_passes=False),
    )
    sc_out = sc(inputs, xs_pad.reshape(-1), ys.reshape(-1),
                lut.reshape(-1), inv.reshape(-1), dy.reshape(-1))
    tc_out = _tc_calibrate(inputs, xs.T, ys.T)
    return jnp.concatenate([sc_out, tc_out], axis=0)


# speedup vs baseline: 1.0926x; 1.0926x over previous
import functools
import jax
import jax.numpy as jnp
from jax import lax
from jax.experimental import pallas as pl
from jax.experimental.pallas import tpu as pltpu
from jax.experimental.pallas import tpu_sc as plsc

BATCH = 16384
N_UNIT = 1024
N_BIN = 50
N_PAD = 64
NC = 2
NS = 16
NW = NC * NS
N_COLG = 8
N_ROWQ = NW // N_COLG
U_PER_W = N_UNIT // N_COLG
B_PER_W = BATCH // N_ROWQ
CHUNK = 256
L = 16
NCELL = 256
SC_ROWS = 10240
SC_B_PER_W = SC_ROWS // N_ROWQ


def _tables_block(xs_ref, ys_ref, lut_ref, inv_ref, dy_ref):
    cells = jax.lax.broadcasted_iota(jnp.int32, (1, NCELL), 1).astype(jnp.float32)
    e_lo = cells * jnp.float32(1.0 / NCELL)
    e_hi = (cells + 1.0) * jnp.float32(1.0 / NCELL)
    cnt_lo = jnp.zeros((N_UNIT, NCELL), jnp.int32)
    cnt_hi = jnp.zeros((N_UNIT, NCELL), jnp.int32)
    one = jnp.int32(1)
    zero = jnp.int32(0)
    for j in range(N_BIN):
        xj = xs_ref[:, j][:, None]
        cnt_lo = cnt_lo + jnp.where(xj <= e_lo, one, zero)
        cnt_hi = cnt_hi + jnp.where(xj <= e_hi, one, zero)
    ulocal = jax.lax.broadcasted_iota(jnp.int32, (N_UNIT, 1), 0) % U_PER_W
    start = ulocal * N_PAD + jnp.maximum(cnt_lo, 1)
    lut_ref[...] = start | ((cnt_hi - cnt_lo) << 16)

    xs = xs_ref[...]
    ys = ys_ref[...]
    x_hi = jnp.concatenate([xs[:, 1:], xs[:, N_BIN - 1:]], axis=1)
    y_hi = jnp.concatenate([ys[:, 1:], ys[:, N_BIN - 1:]], axis=1)
    inv_ref[...] = 1.0 / jnp.maximum(x_hi - xs, jnp.float32(1e-12))
    dy_ref[...] = y_hi - ys


def _build_tables(xs, ys):
    return pl.pallas_call(
        _tables_block,
        out_shape=(
            jax.ShapeDtypeStruct((N_UNIT, NCELL), jnp.int32),
            jax.ShapeDtypeStruct((N_UNIT, N_BIN), jnp.float32),
            jax.ShapeDtypeStruct((N_UNIT, N_BIN), jnp.float32),
        ),
    )(xs, ys)


def _next_down(v):
    bits = jax.lax.bitcast_convert_type(v, jnp.int32)
    dec = jax.lax.bitcast_convert_type(bits - 1, jnp.float32)
    neg_tiny = jnp.float32(-1e-30)
    return jnp.where(v > 0, dec, jnp.minimum(v, neg_tiny) * jnp.float32(1.0000001))


def _isotonic_block(x_ref, xs_ref, ys_ref, o_ref, *, n_bin):
    x = x_ref[...]
    xs_rows = [xs_ref[j, :] for j in range(n_bin)]
    ys_rows = [ys_ref[j, :] for j in range(n_bin)]
    u = [None] * n_bin
    u[n_bin - 1] = xs_rows[n_bin - 1]
    for j in range(n_bin - 2, -1, -1):
        u[j] = jnp.minimum(xs_rows[j], _next_down(u[j + 1]))
    acc = jnp.broadcast_to(ys_rows[0][None, :], x.shape)
    for j in range(n_bin - 1):
        w = u[j + 1] - u[j]
        s = (ys_rows[j + 1] - ys_rows[j]) / w
        t = jnp.minimum(jnp.maximum(x - u[j][None, :], 0.0), w[None, :])
        acc = acc + t * s[None, :]
    lo_mask = x <= xs_rows[0][None, :]
    hi_mask = x >= xs_rows[n_bin - 1][None, :]
    out = jnp.where(lo_mask, ys_rows[0][None, :],
                    jnp.where(hi_mask, ys_rows[n_bin - 1][None, :], acc))
    o_ref[...] = out


def _tc_calibrate(inputs_full, xs_t, ys_t):
    bb = 1024
    off = SC_ROWS // bb
    return pl.pallas_call(
        functools.partial(_isotonic_block, n_bin=N_BIN),
        grid=((BATCH - SC_ROWS) // bb,),
        in_specs=[
            pl.BlockSpec((bb, N_UNIT), lambda i: (i + off, 0)),
            pl.BlockSpec((N_BIN, N_UNIT), lambda i: (0, 0)),
            pl.BlockSpec((N_BIN, N_UNIT), lambda i: (0, 0)),
        ],
        out_specs=pl.BlockSpec((bb, N_UNIT), lambda i: (i, 0)),
        out_shape=jax.ShapeDtypeStruct((BATCH - SC_ROWS, N_UNIT), jnp.float32),
    )(inputs_full, xs_t, ys_t)


def _sc_body(in_hbm, xsp_hbm, ys_hbm, lut_hbm, inv_hbm, dy_hbm, out_hbm,
             xs_v, ys_v, lut_v, inv_v, dy_v, inb, outb):
    wid = lax.axis_index("s") * NC + lax.axis_index("c")
    u0 = (wid // N_ROWQ) * U_PER_W
    r0_base = (wid % N_ROWQ) * SC_B_PER_W

    pltpu.sync_copy(xsp_hbm.at[pl.ds(u0 * N_PAD, U_PER_W * N_PAD)], xs_v)
    pltpu.sync_copy(ys_hbm.at[pl.ds(u0 * N_BIN, U_PER_W * N_BIN)], ys_v)
    pltpu.sync_copy(lut_hbm.at[pl.ds(u0 * NCELL, U_PER_W * NCELL)], lut_v)
    pltpu.sync_copy(inv_hbm.at[pl.ds(u0 * N_BIN, U_PER_W * N_BIN)], inv_v)
    pltpu.sync_copy(dy_hbm.at[pl.ds(u0 * N_BIN, U_PER_W * N_BIN)], dy_v)

    lane = lax.iota(jnp.int32, L)
    lane_x = lane * N_PAD
    lane_l = lane * NCELL
    lane_y = lane * N_BIN
    n_h = U_PER_W // L

    lane_d = lane * (N_BIN - N_PAD)

    def _finish(h, x, rf, xbase, clip_low):
        if clip_low:
            rf = jnp.maximum(rf, xbase + 1)
        xlo_i = jnp.minimum(rf - 1, xbase + (N_BIN - 2))
        ylo_i = xlo_i + (lane_d + h * L * (N_BIN - N_PAD))
        x_lo = plsc.load_gather(xs_v, [xlo_i])
        y_lo = plsc.load_gather(ys_v, [ylo_i])
        inv = plsc.load_gather(inv_v, [ylo_i])
        dy = plsc.load_gather(dy_v, [ylo_i])
        t = jnp.clip((x - x_lo) * inv, jnp.float32(0.0), jnp.float32(1.0))
        return y_lo + t * dy

    zero16 = jnp.zeros((L,), jnp.int32)

    def fast_row(row, flagacc):
        for h in range(n_h):
            x = inb[row, pl.ds(h * L, L)]
            xbase = lane_x + h * (L * N_PAD)
            cell = (x * jnp.float32(NCELL)).astype(jnp.int32)
            lw = plsc.load_gather(lut_v, [lane_l + h * (L * NCELL) + cell])
            rf = lw & 0xFFFF
            gap = lw >> 16
            for step in (4, 2, 1):
                probe = plsc.load_gather(xs_v, [rf + (step - 1)])
                rf = jnp.where(probe <= x, rf + step, rf)
            flagacc = jnp.maximum(flagacc, gap)
            outb[row, pl.ds(h * L, L)] = _finish(h, x, rf, xbase, False)
        return flagacc

    def slow_row(row):
        for h in range(n_h):
            x = inb[row, pl.ds(h * L, L)]
            xbase = lane_x + h * (L * N_PAD)
            rf = xbase
            for step in (32, 16, 8, 4, 2, 1):
                probe = plsc.load_gather(xs_v, [rf + (step - 1)])
                rf = jnp.where(probe <= x, rf + step, rf)
            outb[row, pl.ds(h * L, L)] = _finish(h, x, rf, xbase, True)

    def chunk_body(i, _):
        row0 = r0_base + i * CHUNK
        pltpu.sync_copy(in_hbm.at[pl.ds(row0, CHUNK), pl.ds(u0, U_PER_W)], inb)
        flags = plsc.parallel_loop(0, CHUNK, step=1, unroll=2,
                                   carry=zero16)(fast_row)
        flag_s = jnp.max(flags)

        @pl.when(flag_s > 7)
        def _():
            plsc.parallel_loop(0, CHUNK, step=1, unroll=2)(slow_row)

        pltpu.sync_copy(outb, out_hbm.at[pl.ds(row0, CHUNK), pl.ds(u0, U_PER_W)])
        return _

    lax.fori_loop(0, SC_B_PER_W // CHUNK, chunk_body, None)


@jax.jit
def kernel(inputs, xs, ys):
    xs_pad = jnp.pad(xs, ((0, 0), (0, N_PAD - N_BIN)),
                     constant_values=jnp.finfo(jnp.float32).max)
    lut, inv, dy = _build_tables(xs, ys)
    mesh = plsc.VectorSubcoreMesh(core_axis_name="c", subcore_axis_name="s")
    sc = pl.kernel(
        _sc_body,
        out_type=jax.ShapeDtypeStruct((SC_ROWS, N_UNIT), jnp.float32),
        mesh=mesh,
        scratch_types=[
            pltpu.VMEM((U_PER_W * N_PAD,), jnp.float32),
            pltpu.VMEM((U_PER_W * N_BIN,), jnp.float32),
            pltpu.VMEM((U_PER_W * NCELL,), jnp.int32),
            pltpu.VMEM((U_PER_W * N_BIN,), jnp.float32),
            pltpu.VMEM((U_PER_W * N_BIN,), jnp.float32),
            pltpu.VMEM((CHUNK, U_PER_W), jnp.float32),
            pltpu.VMEM((CHUNK, U_PER_W), jnp.float32),
        ],
        compiler_params=pltpu.CompilerParams(needs_layout_passes=False),
    )
    sc_out = sc(inputs, xs_pad.reshape(-1), ys.reshape(-1),
                lut.reshape(-1), inv.reshape(-1), dy.reshape(-1))
    tc_out = _tc_calibrate(inputs, xs.T, ys.T)
    return jnp.concatenate([sc_out, tc_out], axis=0)


# speedup vs baseline: 1.1985x; 1.0969x over previous
import functools
import jax
import jax.numpy as jnp
from jax import lax
from jax.experimental import pallas as pl
from jax.experimental.pallas import tpu as pltpu
from jax.experimental.pallas import tpu_sc as plsc

BATCH = 16384
N_UNIT = 1024
N_BIN = 50
N_PAD = 64
NC = 2
NS = 16
NW = NC * NS
N_COLG = 8
N_ROWQ = NW // N_COLG
U_PER_W = N_UNIT // N_COLG
B_PER_W = BATCH // N_ROWQ
CHUNK = 256
L = 16
NCELL = 256
SC_ROWS = 11264
SC_B_PER_W = SC_ROWS // N_ROWQ


def _tables_block(xs_ref, ys_ref, lut_ref, inv_ref, dy_ref):
    cells = jax.lax.broadcasted_iota(jnp.int32, (1, NCELL), 1).astype(jnp.float32)
    e_lo = cells * jnp.float32(1.0 / NCELL)
    e_hi = (cells + 1.0) * jnp.float32(1.0 / NCELL)
    cnt_lo = jnp.zeros((N_UNIT, NCELL), jnp.int32)
    cnt_hi = jnp.zeros((N_UNIT, NCELL), jnp.int32)
    one = jnp.int32(1)
    zero = jnp.int32(0)
    for j in range(N_BIN):
        xj = xs_ref[:, j][:, None]
        cnt_lo = cnt_lo + jnp.where(xj <= e_lo, one, zero)
        cnt_hi = cnt_hi + jnp.where(xj <= e_hi, one, zero)
    ulocal = jax.lax.broadcasted_iota(jnp.int32, (N_UNIT, 1), 0) % U_PER_W
    start = ulocal * N_PAD + jnp.maximum(cnt_lo, 1)
    lut_ref[...] = start | ((cnt_hi - cnt_lo) << 16)

    xs = xs_ref[...]
    ys = ys_ref[...]
    x_hi = jnp.concatenate([xs[:, 1:], xs[:, N_BIN - 1:]], axis=1)
    y_hi = jnp.concatenate([ys[:, 1:], ys[:, N_BIN - 1:]], axis=1)
    inv_ref[...] = 1.0 / jnp.maximum(x_hi - xs, jnp.float32(1e-12))
    dy_ref[...] = y_hi - ys


def _build_tables(xs, ys):
    return pl.pallas_call(
        _tables_block,
        out_shape=(
            jax.ShapeDtypeStruct((N_UNIT, NCELL), jnp.int32),
            jax.ShapeDtypeStruct((N_UNIT, N_BIN), jnp.float32),
            jax.ShapeDtypeStruct((N_UNIT, N_BIN), jnp.float32),
        ),
    )(xs, ys)


def _next_down(v):
    bits = jax.lax.bitcast_convert_type(v, jnp.int32)
    dec = jax.lax.bitcast_convert_type(bits - 1, jnp.float32)
    neg_tiny = jnp.float32(-1e-30)
    return jnp.where(v > 0, dec, jnp.minimum(v, neg_tiny) * jnp.float32(1.0000001))


def _isotonic_block(x_ref, xs_ref, ys_ref, o_ref, *, n_bin):
    x = x_ref[...]
    xs_rows = [xs_ref[j, :] for j in range(n_bin)]
    ys_rows = [ys_ref[j, :] for j in range(n_bin)]
    u = [None] * n_bin
    u[n_bin - 1] = xs_rows[n_bin - 1]
    for j in range(n_bin - 2, -1, -1):
        u[j] = jnp.minimum(xs_rows[j], _next_down(u[j + 1]))
    acc = jnp.broadcast_to(ys_rows[0][None, :], x.shape)
    for j in range(n_bin - 1):
        w = u[j + 1] - u[j]
        s = (ys_rows[j + 1] - ys_rows[j]) / w
        t = jnp.minimum(jnp.maximum(x - u[j][None, :], 0.0), w[None, :])
        acc = acc + t * s[None, :]
    lo_mask = x <= xs_rows[0][None, :]
    hi_mask = x >= xs_rows[n_bin - 1][None, :]
    out = jnp.where(lo_mask, ys_rows[0][None, :],
                    jnp.where(hi_mask, ys_rows[n_bin - 1][None, :], acc))
    o_ref[...] = out


def _tc_calibrate(inputs_full, xs_t, ys_t):
    bb = 1024
    off = SC_ROWS // bb
    return pl.pallas_call(
        functools.partial(_isotonic_block, n_bin=N_BIN),
        grid=((BATCH - SC_ROWS) // bb,),
        in_specs=[
            pl.BlockSpec((bb, N_UNIT), lambda i: (i + off, 0)),
            pl.BlockSpec((N_BIN, N_UNIT), lambda i: (0, 0)),
            pl.BlockSpec((N_BIN, N_UNIT), lambda i: (0, 0)),
        ],
        out_specs=pl.BlockSpec((bb, N_UNIT), lambda i: (i, 0)),
        out_shape=jax.ShapeDtypeStruct((BATCH - SC_ROWS, N_UNIT), jnp.float32),
    )(inputs_full, xs_t, ys_t)


def _sc_body(in_hbm, xsp_hbm, ys_hbm, lut_hbm, inv_hbm, dy_hbm, out_hbm,
             xs_v, ys_v, lut_v, inv_v, dy_v, inb, outb):
    wid = lax.axis_index("s") * NC + lax.axis_index("c")
    u0 = (wid // N_ROWQ) * U_PER_W
    r0_base = (wid % N_ROWQ) * SC_B_PER_W

    pltpu.sync_copy(xsp_hbm.at[pl.ds(u0 * N_PAD, U_PER_W * N_PAD)], xs_v)
    pltpu.sync_copy(ys_hbm.at[pl.ds(u0 * N_BIN, U_PER_W * N_BIN)], ys_v)
    pltpu.sync_copy(lut_hbm.at[pl.ds(u0 * NCELL, U_PER_W * NCELL)], lut_v)
    pltpu.sync_copy(inv_hbm.at[pl.ds(u0 * N_BIN, U_PER_W * N_BIN)], inv_v)
    pltpu.sync_copy(dy_hbm.at[pl.ds(u0 * N_BIN, U_PER_W * N_BIN)], dy_v)

    lane = lax.iota(jnp.int32, L)
    lane_x = lane * N_PAD
    lane_l = lane * NCELL
    lane_y = lane * N_BIN
    n_h = U_PER_W // L

    lane_d = lane * (N_BIN - N_PAD)

    def _finish(h, x, rf, xbase, clip_low):
        if clip_low:
            rf = jnp.maximum(rf, xbase + 1)
        xlo_i = jnp.minimum(rf - 1, xbase + (N_BIN - 2))
        ylo_i = xlo_i + (lane_d + h * L * (N_BIN - N_PAD))
        x_lo = plsc.load_gather(xs_v, [xlo_i])
        y_lo = plsc.load_gather(ys_v, [ylo_i])
        inv = plsc.load_gather(inv_v, [ylo_i])
        dy = plsc.load_gather(dy_v, [ylo_i])
        t = jnp.clip((x - x_lo) * inv, jnp.float32(0.0), jnp.float32(1.0))
        return y_lo + t * dy

    zero16 = jnp.zeros((L,), jnp.int32)

    def fast_row(row, flagacc):
        for h in range(n_h):
            x = inb[row, pl.ds(h * L, L)]
            xbase = lane_x + h * (L * N_PAD)
            cell = (x * jnp.float32(NCELL)).astype(jnp.int32)
            lw = plsc.load_gather(lut_v, [lane_l + h * (L * NCELL) + cell])
            rf = lw & 0xFFFF
            gap = lw >> 16
            for step in (4, 2, 1):
                probe = plsc.load_gather(xs_v, [rf + (step - 1)])
                rf = jnp.where(probe <= x, rf + step, rf)
            flagacc = jnp.maximum(flagacc, gap)
            outb[row, pl.ds(h * L, L)] = _finish(h, x, rf, xbase, False)
        return flagacc

    def slow_row(row):
        for h in range(n_h):
            x = inb[row, pl.ds(h * L, L)]
            xbase = lane_x + h * (L * N_PAD)
            rf = xbase
            for step in (32, 16, 8, 4, 2, 1):
                probe = plsc.load_gather(xs_v, [rf + (step - 1)])
                rf = jnp.where(probe <= x, rf + step, rf)
            outb[row, pl.ds(h * L, L)] = _finish(h, x, rf, xbase, True)

    def chunk_body(i, _):
        row0 = r0_base + i * CHUNK
        pltpu.sync_copy(in_hbm.at[pl.ds(row0, CHUNK), pl.ds(u0, U_PER_W)], inb)
        flags = plsc.parallel_loop(0, CHUNK, step=1, unroll=2,
                                   carry=zero16)(fast_row)
        flag_s = jnp.max(flags)

        @pl.when(flag_s > 7)
        def _():
            plsc.parallel_loop(0, CHUNK, step=1, unroll=2)(slow_row)

        pltpu.sync_copy(outb, out_hbm.at[pl.ds(row0, CHUNK), pl.ds(u0, U_PER_W)])
        return _

    lax.fori_loop(0, SC_B_PER_W // CHUNK, chunk_body, None)


@jax.jit
def kernel(inputs, xs, ys):
    xs_pad = jnp.pad(xs, ((0, 0), (0, N_PAD - N_BIN)),
                     constant_values=jnp.finfo(jnp.float32).max)
    lut, inv, dy = _build_tables(xs, ys)
    mesh = plsc.VectorSubcoreMesh(core_axis_name="c", subcore_axis_name="s")
    sc = pl.kernel(
        _sc_body,
        out_type=jax.ShapeDtypeStruct((SC_ROWS, N_UNIT), jnp.float32),
        mesh=mesh,
        scratch_types=[
            pltpu.VMEM((U_PER_W * N_PAD,), jnp.float32),
            pltpu.VMEM((U_PER_W * N_BIN,), jnp.float32),
            pltpu.VMEM((U_PER_W * NCELL,), jnp.int32),
            pltpu.VMEM((U_PER_W * N_BIN,), jnp.float32),
            pltpu.VMEM((U_PER_W * N_BIN,), jnp.float32),
            pltpu.VMEM((CHUNK, U_PER_W), jnp.float32),
            pltpu.VMEM((CHUNK, U_PER_W), jnp.float32),
        ],
        compiler_params=pltpu.CompilerParams(needs_layout_passes=False),
    )
    sc_out = sc(inputs, xs_pad.reshape(-1), ys.reshape(-1),
                lut.reshape(-1), inv.reshape(-1), dy.reshape(-1))
    tc_out = _tc_calibrate(inputs, xs.T, ys.T)
    return jnp.concatenate([sc_out, tc_out], axis=0)
